# baseline (speedup 1.0000x reference)
import functools

import jax
import jax.numpy as jnp
from jax import lax
from jax.experimental import pallas as pl
from jax.experimental.pallas import tpu as pltpu
from jax.experimental.pallas import tpu_sc as plsc

NC = 2
NS = 16
NW = NC * NS
CH = 128
SL = 8
NBUF = 2
IDXD = 4


def _dotT(a, b):
    return lax.dot_general(a, b, (((1,), (1,)), ((), ())),
                           preferred_element_type=jnp.float32)


def _tc_pre_body(x_ref, wl_ref, wr_ref, bl_ref, m_ref, r_ref):
    xb = x_ref[...]
    m_ref[...] = _dotT(xb, wl_ref[...])
    r_ref[...] = _dotT(xb, wr_ref[...]) + bl_ref[...]


def _mean_root(p, cnt_ref, r_ref):
    inv = 1.0 / jnp.maximum(cnt_ref[0, :, 0:1], 1.0)
    return p * inv + r_ref[...]


def _tc_mid1_body(pc_ref, cnt_ref, r_ref, wl_ref, wr_ref, bl_ref,
                  m_ref, rn_ref):
    h = jnp.maximum(_mean_root(pc_ref[0], cnt_ref, r_ref), 0.0)
    m_ref[...] = _dotT(h, wl_ref[...])
    rn_ref[...] = _dotT(h, wr_ref[...]) + bl_ref[...]


def _tc_mid2_body(p_ref, cnt_ref, r_ref, wl_ref, wr_ref, bl_ref,
                  m_ref, rn_ref):
    h = jnp.maximum(_mean_root(p_ref[...], cnt_ref, r_ref), 0.0)
    m_ref[...] = _dotT(h, wl_ref[...])
    rn_ref[...] = _dotT(h, wr_ref[...]) + bl_ref[...]


def _tc_fin_body(p_ref, cnt_ref, r_ref, o_ref):
    o_ref[...] = _mean_root(p_ref[...], cnt_ref, r_ref)


def _make_sc_agg(np_rows, d, n0, with_count):
    mesh = plsc.VectorSubcoreMesh(core_axis_name="c", subcore_axis_name="s")
    rpt = np_rows // NS
    assert n0 % 16 == 0
    nslab = n0 // 16
    out_sds = (jax.ShapeDtypeStruct((NC, np_rows, d), jnp.float32)
               if with_count else
               jax.ShapeDtypeStruct((np_rows, d), jnp.float32))

    @functools.partial(
        pl.kernel,
        out_type=out_sds,
        mesh=mesh,
        scratch_types=[
            pltpu.VMEM_SHARED((np_rows, d), jnp.float32),
            pltpu.VMEM((IDXD, CH), jnp.int32),
            pltpu.VMEM((NBUF, CH), jnp.int32),
            pltpu.VMEM((2, SL, CH), jnp.int32),
            pltpu.VMEM((NBUF, CH, d), jnp.float32),
        ] + [pltpu.SemaphoreType.DMA] * (NBUF + NBUF + IDXD + 1),
    )
    def sc_agg(m_hbm, src_hbm, dst_hbm, z_hbm, ones_hbm, out_hbm,
               acc_sh, srcr_v, dstr_v, slab_v, rows_v, *sems):
        rsem = sems[:NBUF]
        isem = sems[NBUF:2 * NBUF]
        ssem = sems[2 * NBUF:2 * NBUF + IDXD]
        csem = sems[2 * NBUF + IDXD]
        c = lax.axis_index("c")
        s = lax.axis_index("s")
        bslot = s * (n0 // SL)
        r0 = s * rpt
        pltpu.sync_copy(z_hbm.at[pl.ds(r0, rpt)], acc_sh.at[pl.ds(r0, rpt)])
        plsc.subcore_barrier()

        @pl.when(c == 0)
        def _gather_core():
            for k in range(IDXD):
                pltpu.async_copy(src_hbm.at[bslot + k // SL, k % SL],
                                 srcr_v.at[k], ssem[k])
            for b in range(NBUF):
                pltpu.async_copy(dst_hbm.at[bslot + b // SL, b % SL],
                                 dstr_v.at[b], isem[b])
                pltpu.make_async_copy(src_hbm.at[bslot, b],
                                      srcr_v.at[b], ssem[b]).wait()
                pltpu.async_copy(m_hbm.at[srcr_v.at[b]], rows_v.at[b],
                                 rsem[b])

            def quad(u, carry):
                j0 = u * 4
                for q in range(4):
                    j = j0 + q
                    b = q % 2
                    pltpu.make_async_copy(dst_hbm.at[bslot, q],
                                          dstr_v.at[b], isem[b]).wait()
                    pltpu.make_async_copy(m_hbm.at[srcr_v.at[q]],
                                          rows_v.at[b], rsem[b]).wait()
                    pltpu.sync_copy(rows_v.at[b], acc_sh.at[dstr_v.at[b]],
                                    add=True)
                    jf = j + IDXD

                    @pl.when(jf < n0)
                    def _():
                        pltpu.async_copy(
                            src_hbm.at[bslot + jf // SL, jf % SL],
                            srcr_v.at[q], ssem[q])
                    jn = j + NBUF

                    @pl.when(jn < n0)
                    def _():
                        pltpu.async_copy(
                            dst_hbm.at[bslot + jn // SL, jn % SL],
                            dstr_v.at[b], isem[b])
                        pltpu.make_async_copy(
                            src_hbm.at[bslot, b],
                            srcr_v.at[(q + NBUF) % IDXD],
                            ssem[(q + NBUF) % IDXD]).wait()
                        pltpu.async_copy(m_hbm.at[srcr_v.at[(q + NBUF) % IDXD]],
                                         rows_v.at[b], rsem[b])
                return carry

            lax.fori_loop(0, n0 // 4, quad, 0, unroll=False)

        if with_count:
            @pl.when(c == 1)
            def _count_core():
                pltpu.sync_copy(ones_hbm, rows_v.at[0])

                def slab(k, carry):
                    pltpu.sync_copy(dst_hbm.at[pl.ds(bslot + 2 * k, 2)],
                                    slab_v)
                    for q in range(16):
                        pltpu.async_copy(
                            rows_v.at[0],
                            acc_sh.at[slab_v.at[q // SL, q % SL]],
                            csem, add=True)
                    for q in range(16):
                        pltpu.make_async_copy(
                            rows_v.at[0],
                            acc_sh.at[slab_v.at[q // SL, q % SL]],
                            csem).wait()
                    return carry

                lax.fori_loop(0, nslab, slab, 0, unroll=False)

        plsc.subcore_barrier()
        if with_count:
            pltpu.sync_copy(acc_sh.at[pl.ds(r0, rpt)],
                            out_hbm.at[c, pl.ds(r0, rpt)])
        else:
            @pl.when(c == 0)
            def _out():
                pltpu.sync_copy(acc_sh.at[pl.ds(r0, rpt)],
                                out_hbm.at[pl.ds(r0, rpt)])

    return sc_agg


def kernel(x, edge_index, Wl0, bl0, Wr0, Wl1, bl1, Wr1, Wl2, bl2, Wr2):
    n, d = x.shape
    e = edge_index.shape[1]

    n0 = -(-e // (NS * CH * 16)) * 16
    nslots = NS * n0 // SL
    np_rows = -(-(n + 8) // (8 * NS)) * (8 * NS)
    blk = 1024
    grid = (-(-n // blk),)

    src = edge_index[0]
    dst = edge_index[1]
    pad = nslots * SL * CH - e
    src_p = jnp.concatenate(
        [src, jnp.zeros((pad,), jnp.int32)]).reshape(nslots, SL, CH)
    dst_p = jnp.concatenate(
        [dst, jnp.full((pad,), n, jnp.int32)]).reshape(nslots, SL, CH)
    zeros_d = jnp.zeros((np_rows, d), jnp.float32)
    ones_c = jnp.ones((CH, d), jnp.float32)

    sc_agg0 = _make_sc_agg(np_rows, d, n0, True)
    sc_agg = _make_sc_agg(np_rows, d, n0, False)

    w_spec = pl.BlockSpec((d, d), lambda i: (0, 0))
    b_spec = pl.BlockSpec((1, d), lambda i: (0, 0))
    h_spec = pl.BlockSpec((blk, d), lambda i: (i, 0))
    p_spec = pl.BlockSpec((blk, d), lambda i: (i, 0))
    s_spec = pl.BlockSpec((1, blk, d), lambda i: (0, i, 0))
    c_spec = pl.BlockSpec((1, blk, d), lambda i: (1, i, 0))
    h_sds = jax.ShapeDtypeStruct((n, d), jnp.float32)

    tc_pre = pl.pallas_call(
        _tc_pre_body, grid=grid,
        in_specs=[h_spec, w_spec, w_spec, b_spec],
        out_specs=[h_spec, h_spec],
        out_shape=[h_sds, h_sds],
    )
    tc_mid1 = pl.pallas_call(
        _tc_mid1_body, grid=grid,
        in_specs=[s_spec, c_spec, h_spec, w_spec, w_spec, b_spec],
        out_specs=[h_spec, h_spec],
        out_shape=[h_sds, h_sds],
    )
    tc_mid2 = pl.pallas_call(
        _tc_mid2_body, grid=grid,
        in_specs=[p_spec, c_spec, h_spec, w_spec, w_spec, b_spec],
        out_specs=[h_spec, h_spec],
        out_shape=[h_sds, h_sds],
    )
    tc_fin = pl.pallas_call(
        _tc_fin_body, grid=grid,
        in_specs=[p_spec, c_spec, h_spec],
        out_specs=h_spec,
        out_shape=h_sds,
    )

    m, r = tc_pre(x, Wl0, Wr0, bl0.reshape(1, d))
    pc = sc_agg0(m, src_p, dst_p, zeros_d, ones_c)
    m, r = tc_mid1(pc, pc, r, Wl1, Wr1, bl1.reshape(1, d))
    p = sc_agg(m, src_p, dst_p, zeros_d, ones_c)
    m, r = tc_mid2(p, pc, r, Wl2, Wr2, bl2.reshape(1, d))
    p = sc_agg(m, src_p, dst_p, zeros_d, ones_c)
    return tc_fin(p, pc, r)


# speedup vs baseline: 1.0407x; 1.0407x over previous
import functools

import jax
import jax.numpy as jnp
from jax import lax
from jax.experimental import pallas as pl
from jax.experimental.pallas import tpu as pltpu
from jax.experimental.pallas import tpu_sc as plsc

NC = 2
NS = 16
NW = NC * NS
CH = 128
SL = 8
NBUF = 2
IDXD = 4


def _dotT(a, b):
    return lax.dot_general(a, b, (((1,), (1,)), ((), ())),
                           preferred_element_type=jnp.float32)


def _tc_pre_body(x_ref, wl_ref, wr_ref, bl_ref, m_ref, r_ref):
    xb = x_ref[...]
    m_ref[...] = _dotT(xb, wl_ref[...])
    r_ref[...] = _dotT(xb, wr_ref[...]) + bl_ref[...]


def _mean_root(p, cnt_ref, r_ref):
    inv = 1.0 / jnp.maximum(cnt_ref[0, :, 0:1], 1.0)
    return p * inv + r_ref[...]


def _tc_mid1_body(pc_ref, cnt_ref, r_ref, wl_ref, wr_ref, bl_ref,
                  m_ref, rn_ref):
    h = jnp.maximum(_mean_root(pc_ref[0], cnt_ref, r_ref), 0.0)
    m_ref[...] = _dotT(h, wl_ref[...])
    rn_ref[...] = _dotT(h, wr_ref[...]) + bl_ref[...]


def _tc_mid2_body(p_ref, cnt_ref, r_ref, wl_ref, wr_ref, bl_ref,
                  m_ref, rn_ref):
    h = jnp.maximum(_mean_root(p_ref[...], cnt_ref, r_ref), 0.0)
    m_ref[...] = _dotT(h, wl_ref[...])
    rn_ref[...] = _dotT(h, wr_ref[...]) + bl_ref[...]


def _tc_fin_body(p_ref, cnt_ref, r_ref, o_ref):
    o_ref[...] = _mean_root(p_ref[...], cnt_ref, r_ref)


def _make_sc_agg(np_rows, d, n0, with_count):
    mesh = plsc.VectorSubcoreMesh(core_axis_name="c", subcore_axis_name="s")
    rpt = np_rows // NS
    assert n0 % 16 == 0
    hc = n0 // 2
    hcs = hc // SL
    assert hc % NBUF == 0 and hc // NBUF >= 2
    out_sds = (jax.ShapeDtypeStruct((NC, np_rows, d), jnp.float32)
               if with_count else
               jax.ShapeDtypeStruct((np_rows, d), jnp.float32))

    @functools.partial(
        pl.kernel,
        out_type=out_sds,
        mesh=mesh,
        scratch_types=[
            pltpu.VMEM_SHARED((np_rows, d), jnp.float32),
            pltpu.VMEM((hcs, SL, CH), jnp.int32),
            pltpu.VMEM((NBUF, CH), jnp.int32),
            pltpu.VMEM((NBUF, CH, d), jnp.float32),
        ] + [pltpu.SemaphoreType.DMA] * (2 * NBUF + 1),
    )
    def sc_agg(m_hbm, src_hbm, dst_hbm, z_hbm, ones_hbm, out_hbm,
               acc_sh, idx_v, dstr_v, rows_v, *sems):
        rsem = sems[:NBUF]
        isem = sems[NBUF:2 * NBUF]
        csem = sems[2 * NBUF]
        c = lax.axis_index("c")
        s = lax.axis_index("s")
        r0 = s * rpt
        pltpu.sync_copy(z_hbm.at[pl.ds(r0, rpt)], acc_sh.at[pl.ds(r0, rpt)])
        plsc.subcore_barrier()

        @pl.when(c == 0)
        def _gather_core():
            for half in range(2):
                bslot = s * (n0 // SL) + half * hcs
                pltpu.sync_copy(src_hbm.at[pl.ds(bslot, hcs)], idx_v)
                for b in range(NBUF):
                    pltpu.async_copy(dst_hbm.at[bslot + b // SL, b % SL],
                                     dstr_v.at[b], isem[b])
                    pltpu.async_copy(m_hbm.at[idx_v.at[b // SL, b % SL]],
                                     rows_v.at[b], rsem[b])

                def step(g, carry):
                    j0 = g * NBUF
                    for b in range(NBUF):
                        jj = j0 + b
                        pltpu.make_async_copy(dst_hbm.at[bslot, b],
                                              dstr_v.at[b], isem[b]).wait()
                        pltpu.make_async_copy(
                            m_hbm.at[idx_v.at[0, b]],
                            rows_v.at[b], rsem[b]).wait()
                        pltpu.sync_copy(rows_v.at[b],
                                        acc_sh.at[dstr_v.at[b]], add=True)
                        jn = jj + NBUF
                        pltpu.async_copy(
                            dst_hbm.at[bslot + jn // SL, jn % SL],
                            dstr_v.at[b], isem[b])
                        pltpu.async_copy(
                            m_hbm.at[idx_v.at[jn // SL, jn % SL]],
                            rows_v.at[b], rsem[b])
                    return carry

                lax.fori_loop(0, hc // NBUF - 1, step, 0, unroll=False)
                for b in range(NBUF):
                    pltpu.make_async_copy(dst_hbm.at[bslot, b],
                                          dstr_v.at[b], isem[b]).wait()
                    pltpu.make_async_copy(m_hbm.at[idx_v.at[0, b]],
                                          rows_v.at[b], rsem[b]).wait()
                    pltpu.sync_copy(rows_v.at[b],
                                    acc_sh.at[dstr_v.at[b]], add=True)

        if with_count:
            @pl.when(c == 1)
            def _count_core():
                pltpu.sync_copy(ones_hbm, rows_v.at[0])
                for half in range(2):
                    bslot = s * (n0 // SL) + half * hcs
                    pltpu.sync_copy(dst_hbm.at[pl.ds(bslot, hcs)], idx_v)

                    def fire(j, carry):
                        pltpu.async_copy(
                            rows_v.at[0],
                            acc_sh.at[idx_v.at[j // SL, j % SL]],
                            csem, add=True)
                        return carry

                    lax.fori_loop(0, hc, fire, 0, unroll=False)

                    def drain(j, carry):
                        pltpu.make_async_copy(
                            rows_v.at[0],
                            acc_sh.at[idx_v.at[j // SL, j % SL]],
                            csem).wait()
                        return carry

                    lax.fori_loop(0, hc, drain, 0, unroll=False)

        plsc.subcore_barrier()
        if with_count:
            pltpu.sync_copy(acc_sh.at[pl.ds(r0, rpt)],
                            out_hbm.at[c, pl.ds(r0, rpt)])
        else:
            @pl.when(c == 0)
            def _out():
                pltpu.sync_copy(acc_sh.at[pl.ds(r0, rpt)],
                                out_hbm.at[pl.ds(r0, rpt)])

    return sc_agg


def kernel(x, edge_index, Wl0, bl0, Wr0, Wl1, bl1, Wr1, Wl2, bl2, Wr2):
    n, d = x.shape
    e = edge_index.shape[1]

    n0 = -(-e // (NS * CH * 16)) * 16
    nslots = NS * n0 // SL
    np_rows = -(-(n + 8) // (8 * NS)) * (8 * NS)
    blk = 1024
    grid = (-(-n // blk),)

    src = edge_index[0]
    dst = edge_index[1]
    pad = nslots * SL * CH - e
    src_p = jnp.concatenate(
        [src, jnp.zeros((pad,), jnp.int32)]).reshape(nslots, SL, CH)
    dst_p = jnp.concatenate(
        [dst, jnp.full((pad,), n, jnp.int32)]).reshape(nslots, SL, CH)
    zeros_d = jnp.zeros((np_rows, d), jnp.float32)
    ones_c = jnp.ones((CH, d), jnp.float32)

    sc_agg0 = _make_sc_agg(np_rows, d, n0, True)
    sc_agg = _make_sc_agg(np_rows, d, n0, False)

    w_spec = pl.BlockSpec((d, d), lambda i: (0, 0))
    b_spec = pl.BlockSpec((1, d), lambda i: (0, 0))
    h_spec = pl.BlockSpec((blk, d), lambda i: (i, 0))
    p_spec = pl.BlockSpec((blk, d), lambda i: (i, 0))
    s_spec = pl.BlockSpec((1, blk, d), lambda i: (0, i, 0))
    c_spec = pl.BlockSpec((1, blk, d), lambda i: (1, i, 0))
    h_sds = jax.ShapeDtypeStruct((n, d), jnp.float32)

    tc_pre = pl.pallas_call(
        _tc_pre_body, grid=grid,
        in_specs=[h_spec, w_spec, w_spec, b_spec],
        out_specs=[h_spec, h_spec],
        out_shape=[h_sds, h_sds],
    )
    tc_mid1 = pl.pallas_call(
        _tc_mid1_body, grid=grid,
        in_specs=[s_spec, c_spec, h_spec, w_spec, w_spec, b_spec],
        out_specs=[h_spec, h_spec],
        out_shape=[h_sds, h_sds],
    )
    tc_mid2 = pl.pallas_call(
        _tc_mid2_body, grid=grid,
        in_specs=[p_spec, c_spec, h_spec, w_spec, w_spec, b_spec],
        out_specs=[h_spec, h_spec],
        out_shape=[h_sds, h_sds],
    )
    tc_fin = pl.pallas_call(
        _tc_fin_body, grid=grid,
        in_specs=[p_spec, c_spec, h_spec],
        out_specs=h_spec,
        out_shape=h_sds,
    )

    m, r = tc_pre(x, Wl0, Wr0, bl0.reshape(1, d))
    pc = sc_agg0(m, src_p, dst_p, zeros_d, ones_c)
    m, r = tc_mid1(pc, pc, r, Wl1, Wr1, bl1.reshape(1, d))
    p = sc_agg(m, src_p, dst_p, zeros_d, ones_c)
    m, r = tc_mid2(p, pc, r, Wl2, Wr2, bl2.reshape(1, d))
    p = sc_agg(m, src_p, dst_p, zeros_d, ones_c)
    return tc_fin(p, pc, r)


# speedup vs baseline: 1.4596x; 1.4025x over previous
import functools

import jax
import jax.numpy as jnp
from jax import lax
from jax.experimental import pallas as pl
from jax.experimental.pallas import tpu as pltpu
from jax.experimental.pallas import tpu_sc as plsc

NC = 2
NS = 16
NW = NC * NS
LANES = 16
CH = 128


def _dotT(a, b):
    return lax.dot_general(a, b, (((1,), (1,)), ((), ())),
                           preferred_element_type=jnp.float32)


def _tc_pre_body(x_ref, wl_ref, wr_ref, bl_ref, m_ref, r_ref):
    xb = x_ref[...]
    m_ref[...] = _dotT(xb, wl_ref[...])
    r_ref[...] = _dotT(xb, wr_ref[...]) + bl_ref[...]


def _tc_mid_body(p_ref, cnt_ref, r_ref, wl_ref, wr_ref, bl_ref, m_ref, rn_ref):
    cb = cnt_ref[...]
    cnt = cb[0, :, 0:1] + cb[1, :, 0:1]
    inv = 1.0 / jnp.maximum(cnt, 1.0)
    h = (p_ref[0] + p_ref[1]) * inv + r_ref[...]
    h = jnp.maximum(h, 0.0)
    m_ref[...] = _dotT(h, wl_ref[...])
    rn_ref[...] = _dotT(h, wr_ref[...]) + bl_ref[...]


def _tc_fin_body(p_ref, cnt_ref, r_ref, o_ref):
    cb = cnt_ref[...]
    cnt = cb[0, :, 0:1] + cb[1, :, 0:1]
    inv = 1.0 / jnp.maximum(cnt, 1.0)
    o_ref[...] = (p_ref[0] + p_ref[1]) * inv + r_ref[...]


def _make_sc_agg(n_nodes, np_rows, d, nchunk):
    mesh = plsc.VectorSubcoreMesh(core_axis_name="c", subcore_axis_name="s")
    rpt = np_rows // NS

    @functools.partial(
        pl.kernel,
        out_type=jax.ShapeDtypeStruct((NC, np_rows, d), jnp.float32),
        mesh=mesh,
        scratch_types=[
            pltpu.VMEM_SHARED((np_rows, d), jnp.float32),
            pltpu.VMEM((nchunk, CH), jnp.int32),
            pltpu.VMEM((nchunk, CH), jnp.int32),
            pltpu.VMEM((CH, d), jnp.float32),
            pltpu.SemaphoreType.DMA,
        ],
    )
    def sc_agg(m_hbm, src_hbm, dst_hbm, z_hbm, out_hbm,
               acc_sh, src_v, dst_v, rows_v, sem):
        c = lax.axis_index("c")
        s = lax.axis_index("s")
        wid = s * NC + c
        r0 = s * rpt
        pltpu.sync_copy(z_hbm.at[pl.ds(r0, rpt)], acc_sh.at[pl.ds(r0, rpt)])
        pltpu.sync_copy(src_hbm.at[wid], src_v)
        pltpu.sync_copy(dst_hbm.at[wid], dst_v)
        plsc.subcore_barrier()

        def step(j, carry):
            pltpu.async_copy(m_hbm.at[src_v.at[j]], rows_v, sem).wait()
            pltpu.sync_copy(rows_v, acc_sh.at[dst_v.at[j]], add=True)
            return carry

        lax.fori_loop(0, nchunk, step, 0, unroll=False)
        plsc.subcore_barrier()
        pltpu.sync_copy(acc_sh.at[pl.ds(r0, rpt)],
                        out_hbm.at[c, pl.ds(r0, rpt)])

    return sc_agg


def _make_sc_cnt(np_rows, d, nchunk):
    mesh = plsc.VectorSubcoreMesh(core_axis_name="c", subcore_axis_name="s")
    rpt = np_rows // NS

    @functools.partial(
        pl.kernel,
        out_type=jax.ShapeDtypeStruct((NC, np_rows, d), jnp.float32),
        mesh=mesh,
        scratch_types=[
            pltpu.VMEM_SHARED((np_rows, d), jnp.float32),
            pltpu.VMEM((nchunk, CH), jnp.int32),
            pltpu.VMEM((CH, d), jnp.float32),
        ],
    )
    def sc_cnt(dst_hbm, ones_hbm, z_hbm, out_hbm, cnt_sh, dst_v, ones_v):
        c = lax.axis_index("c")
        s = lax.axis_index("s")
        wid = s * NC + c
        r0 = s * rpt
        pltpu.sync_copy(z_hbm.at[pl.ds(r0, rpt)], cnt_sh.at[pl.ds(r0, rpt)])
        pltpu.sync_copy(dst_hbm.at[wid], dst_v)
        pltpu.sync_copy(ones_hbm, ones_v)
        plsc.subcore_barrier()

        def step(j, carry):
            pltpu.sync_copy(ones_v, cnt_sh.at[dst_v.at[j]], add=True)
            return carry

        lax.fori_loop(0, nchunk, step, 0, unroll=False)
        plsc.subcore_barrier()
        pltpu.sync_copy(cnt_sh.at[pl.ds(r0, rpt)],
                        out_hbm.at[c, pl.ds(r0, rpt)])

    return sc_cnt


def kernel(x, edge_index, Wl0, bl0, Wr0, Wl1, bl1, Wr1, Wl2, bl2, Wr2):
    n, d = x.shape
    e = edge_index.shape[1]

    ew = -(-e // (NW * CH)) * CH
    ep = ew * NW
    nchunk = ew // CH
    np_rows = -(-(n + LANES) // 1024) * 1024
    blk = 1024
    grid = (-(-n // blk),)

    src = edge_index[0]
    dst = edge_index[1]
    pad = ep - e
    src_p = jnp.concatenate([src, jnp.zeros((pad,), jnp.int32)]).reshape(NW, nchunk, CH)
    dst_p = jnp.concatenate([dst, jnp.full((pad,), n, jnp.int32)]).reshape(NW, nchunk, CH)
    zeros_d = jnp.zeros((np_rows, d), jnp.float32)
    ones_c = jnp.ones((CH, d), jnp.float32)

    sc_agg = _make_sc_agg(n, np_rows, d, nchunk)
    sc_cnt = _make_sc_cnt(np_rows, d, nchunk)

    w_spec = pl.BlockSpec((d, d), lambda i: (0, 0))
    b_spec = pl.BlockSpec((1, d), lambda i: (0, 0))
    h_spec = pl.BlockSpec((blk, d), lambda i: (i, 0))
    p_spec = pl.BlockSpec((NC, blk, d), lambda i: (0, i, 0))
    c_spec = p_spec
    h_sds = jax.ShapeDtypeStruct((n, d), jnp.float32)

    tc_pre = pl.pallas_call(
        _tc_pre_body, grid=grid,
        in_specs=[h_spec, w_spec, w_spec, b_spec],
        out_specs=[h_spec, h_spec],
        out_shape=[h_sds, h_sds],
    )
    tc_mid = pl.pallas_call(
        _tc_mid_body, grid=grid,
        in_specs=[p_spec, c_spec, h_spec, w_spec, w_spec, b_spec],
        out_specs=[h_spec, h_spec],
        out_shape=[h_sds, h_sds],
    )
    tc_fin = pl.pallas_call(
        _tc_fin_body, grid=grid,
        in_specs=[p_spec, c_spec, h_spec],
        out_specs=h_spec,
        out_shape=h_sds,
    )

    cnt = sc_cnt(dst_p, ones_c, zeros_d)

    m, r = tc_pre(x, Wl0, Wr0, bl0.reshape(1, d))
    p = sc_agg(m, src_p, dst_p, zeros_d)
    m, r = tc_mid(p, cnt, r, Wl1, Wr1, bl1.reshape(1, d))
    p = sc_agg(m, src_p, dst_p, zeros_d)
    m, r = tc_mid(p, cnt, r, Wl2, Wr2, bl2.reshape(1, d))
    p = sc_agg(m, src_p, dst_p, zeros_d)
    return tc_fin(p, cnt, r)
